# Initial kernel scaffold; baseline (speedup 1.0000x reference)
import functools

import jax
import jax.numpy as jnp
from jax import lax
from jax.experimental import pallas as pl
from jax.experimental.pallas import tpu as pltpu
from jax.experimental.pallas import tpu_sc as plsc

L = 200
B = 1024
M = 64
ROWS = L * B
NW = 32
ROWS_PER_W = ROWS // NW
BLK = 128
NBLK = ROWS_PER_W // BLK
ZBLKS = B // BLK


def _body(W_emb, W_re, pos_emb, il, ip, ir, ie, zeros_hbm,
          out_l, out_p, out_r, out_e, idx_v, rows_v, sem):
    cid = lax.axis_index("c")
    sid = lax.axis_index("s")
    wid = sid * 2 + cid
    start = jnp.where(wid == 0, ZBLKS, 0)

    for k, (table, idx, out) in enumerate(
        ((W_emb, il, out_l), (pos_emb, ip, out_p),
         (W_emb, ir, out_r), (W_re, ie, out_e))):
        pltpu.sync_copy(idx.at[pl.ds(wid * NBLK, NBLK)], idx_v.at[k])

        def body(j, carry, table=table, out=out, k=k):
            pltpu.async_copy(table.at[idx_v.at[k].at[j]], rows_v, sem).wait()
            pltpu.sync_copy(rows_v, out.at[pl.ds(wid * ROWS_PER_W + j * BLK, BLK)])
            return carry

        lax.fori_loop(start, NBLK, body, 0)

    @pl.when(wid == 0)
    def _():
        for out in (out_l, out_p, out_r, out_e):
            pltpu.sync_copy(zeros_hbm, out.at[pl.ds(0, B)])


@jax.jit
def kernel(ly, lp, ry, re, W_emb, W_re, pos_emb):
    zrow = jnp.zeros((1, B), jnp.int32)
    il = jnp.concatenate([zrow, ly[:-1].astype(jnp.int32)], 0).reshape(ROWS // BLK, BLK)
    ip = jnp.concatenate([zrow, lp[:-1].astype(jnp.int32)], 0).reshape(ROWS // BLK, BLK)
    ir = jnp.concatenate([zrow, ry[1:].astype(jnp.int32)], 0).reshape(ROWS // BLK, BLK)
    ie = jnp.concatenate([zrow, re[1:].astype(jnp.int32)], 0).reshape(ROWS // BLK, BLK)
    zeros_hbm = jnp.zeros((B, M), jnp.float32)

    mesh = plsc.VectorSubcoreMesh(core_axis_name="c", subcore_axis_name="s")
    f = pl.kernel(
        _body,
        out_type=[jax.ShapeDtypeStruct((ROWS, M), jnp.float32)] * 4,
        mesh=mesh,
        scratch_types=[
            pltpu.VMEM((4, NBLK, BLK), jnp.int32),
            pltpu.VMEM((BLK, M), jnp.float32),
            pltpu.SemaphoreType.DMA,
        ],
    )
    out_l, out_p, out_r, out_e = f(W_emb, W_re, pos_emb, il, ip, ir, ie, zeros_hbm)
    shp = (L, B, M)
    return (out_l.reshape(shp), out_p.reshape(shp),
            out_r.reshape(shp), out_e.reshape(shp))


# speedup vs baseline: 1.4842x; 1.4842x over previous
import functools

import jax
import jax.numpy as jnp
from jax import lax
from jax.experimental import pallas as pl
from jax.experimental.pallas import tpu as pltpu
from jax.experimental.pallas import tpu_sc as plsc

L = 200
B = 1024
M = 64
ROWS = L * B
NW = 32
ROWS_PER_W = ROWS // NW
BLK = 128
NBLK = ROWS_PER_W // BLK
ZBLKS = B // BLK


def _body(W_emb, W_re, pos_emb, il, ip, ir, ie, zeros_hbm,
          out_l, out_p, out_r, out_e, idx_v, rows_v, sem):
    cid = lax.axis_index("c")
    sid = lax.axis_index("s")
    wid = sid * 2 + cid
    start = jnp.where(wid == 0, ZBLKS, 0)

    for k, (table, idx, out) in enumerate(
        ((W_emb, il, out_l), (pos_emb, ip, out_p),
         (W_emb, ir, out_r), (W_re, ie, out_e))):
        pltpu.sync_copy(idx.at[pl.ds(wid * ROWS_PER_W, ROWS_PER_W)], idx_v.at[k])

        def body(j, carry, table=table, out=out, k=k):
            pltpu.async_copy(
                table.at[idx_v.at[k].at[pl.ds(j * BLK, BLK)]], rows_v, sem
            ).wait()
            pltpu.sync_copy(rows_v, out.at[pl.ds(wid * ROWS_PER_W + j * BLK, BLK)])
            return carry

        lax.fori_loop(start, NBLK, body, 0)

    @pl.when(wid == 0)
    def _():
        for out in (out_l, out_p, out_r, out_e):
            pltpu.sync_copy(zeros_hbm, out.at[pl.ds(0, B)])


@jax.jit
def kernel(ly, lp, ry, re, W_emb, W_re, pos_emb):
    zrow = jnp.zeros((1, B), jnp.int32)
    il = jnp.concatenate([zrow, ly[:-1].astype(jnp.int32)], 0).reshape(ROWS)
    ip = jnp.concatenate([zrow, lp[:-1].astype(jnp.int32)], 0).reshape(ROWS)
    ir = jnp.concatenate([zrow, ry[1:].astype(jnp.int32)], 0).reshape(ROWS)
    ie = jnp.concatenate([zrow, re[1:].astype(jnp.int32)], 0).reshape(ROWS)
    zeros_hbm = jnp.zeros((B, M), jnp.float32)

    mesh = plsc.VectorSubcoreMesh(core_axis_name="c", subcore_axis_name="s")
    f = pl.kernel(
        _body,
        out_type=[jax.ShapeDtypeStruct((ROWS, M), jnp.float32)] * 4,
        mesh=mesh,
        scratch_types=[
            pltpu.VMEM((4, ROWS_PER_W), jnp.int32),
            pltpu.VMEM((BLK, M), jnp.float32),
            pltpu.SemaphoreType.DMA,
        ],
        compiler_params=pltpu.CompilerParams(use_tc_tiling_on_sc=False),
    )
    out_l, out_p, out_r, out_e = f(W_emb, W_re, pos_emb, il, ip, ir, ie, zeros_hbm)
    shp = (L, B, M)
    return (out_l.reshape(shp), out_p.reshape(shp),
            out_r.reshape(shp), out_e.reshape(shp))


# speedup vs baseline: 1.5834x; 1.0669x over previous
import jax
import jax.numpy as jnp
from jax import lax
from jax.experimental import pallas as pl
from jax.experimental.pallas import tpu as pltpu
from jax.experimental.pallas import tpu_sc as plsc

L = 200
B = 1024
M = 64
ROWS = L * B
NW = 32
ROWS_PER_W = ROWS // NW
BLK = 128
NBLK = ROWS_PER_W // BLK
ZBLKS = B // BLK


def _body(W_emb, W_re, pos_emb, il, ip, ir, ie, zeros_hbm,
          out_l, out_p, out_r, out_e, idx_v, rows_v, sem_g, sem_w):
    cid = lax.axis_index("c")
    sid = lax.axis_index("s")
    wid = sid * 2 + cid
    start = jnp.where(wid == 0, ZBLKS, 0)

    for k, (table, idx, out) in enumerate(
        ((W_emb, il, out_l), (pos_emb, ip, out_p),
         (W_emb, ir, out_r), (W_re, ie, out_e))):
        pltpu.sync_copy(idx.at[pl.ds(wid * ROWS_PER_W, ROWS_PER_W)], idx_v.at[k])

        def fire_gather(j, b, table=table, k=k):
            pltpu.async_copy(
                table.at[idx_v.at[k].at[pl.ds(j * BLK, BLK)]],
                rows_v.at[b], sem_g)

        def fire_write(j, b, out=out):
            pltpu.async_copy(
                rows_v.at[b], out.at[pl.ds(wid * ROWS_PER_W + j * BLK, BLK)],
                sem_w)

        def drain_gather(b, out=out):
            pltpu.make_async_copy(out.at[pl.ds(0, BLK)], rows_v.at[b], sem_g).wait()

        def drain_write(b, out=out):
            pltpu.make_async_copy(rows_v.at[b], out.at[pl.ds(0, BLK)], sem_w).wait()

        fire_gather(start, start % 2)

        def body(j, carry):
            b = j % 2

            @pl.when(j >= start + 1)
            def _():
                drain_write((j - 1) % 2)

            @pl.when(j + 1 < NBLK)
            def _():
                fire_gather(j + 1, (j + 1) % 2)

            drain_gather(b)
            fire_write(j, b)
            return carry

        lax.fori_loop(start, NBLK, body, 0)
        drain_write((NBLK - 1) % 2)

    @pl.when(wid == 0)
    def _():
        for out in (out_l, out_p, out_r, out_e):
            pltpu.sync_copy(zeros_hbm, out.at[pl.ds(0, B)])


@jax.jit
def kernel(ly, lp, ry, re, W_emb, W_re, pos_emb):
    zrow = jnp.zeros((1, B), jnp.int32)
    il = jnp.concatenate([zrow, ly[:-1].astype(jnp.int32)], 0).reshape(ROWS)
    ip = jnp.concatenate([zrow, lp[:-1].astype(jnp.int32)], 0).reshape(ROWS)
    ir = jnp.concatenate([zrow, ry[1:].astype(jnp.int32)], 0).reshape(ROWS)
    ie = jnp.concatenate([zrow, re[1:].astype(jnp.int32)], 0).reshape(ROWS)
    zeros_hbm = jnp.zeros((B, M), jnp.float32)

    mesh = plsc.VectorSubcoreMesh(core_axis_name="c", subcore_axis_name="s")
    f = pl.kernel(
        _body,
        out_type=[jax.ShapeDtypeStruct((ROWS, M), jnp.float32)] * 4,
        mesh=mesh,
        scratch_types=[
            pltpu.VMEM((4, ROWS_PER_W), jnp.int32),
            pltpu.VMEM((2, BLK, M), jnp.float32),
            pltpu.SemaphoreType.DMA,
            pltpu.SemaphoreType.DMA,
        ],
        compiler_params=pltpu.CompilerParams(use_tc_tiling_on_sc=False),
    )
    out_l, out_p, out_r, out_e = f(W_emb, W_re, pos_emb, il, ip, ir, ie, zeros_hbm)
    shp = (L, B, M)
    return (out_l.reshape(shp), out_p.reshape(shp),
            out_r.reshape(shp), out_e.reshape(shp))
